# Initial kernel scaffold; baseline (speedup 1.0000x reference)
import jax
import jax.numpy as jnp
from jax.experimental import pallas as pl


def kernel(x, Wg, We):
    raise NotImplementedError("write your pallas kernel here")


# speedup vs baseline: 5.6382x; 5.6382x over previous
import functools

import jax
import jax.numpy as jnp
from jax import lax
from jax.experimental import pallas as pl
from jax.experimental.pallas import tpu as pltpu
from jax.experimental.pallas import tpu_sc as plsc

B_, L_, D_ = 4, 2048, 1024
E_, K_ = 64, 2
T_ = B_ * L_
N_ = T_ * K_

TG = 512


def _gate_kernel(x_ref, wg_ref, g_ref, i_ref, r_ref, c_ref):
    s = pl.program_id(0)
    logits = jnp.dot(x_ref[...], wg_ref[...], preferred_element_type=jnp.float32)
    sig = jax.nn.sigmoid(logits)
    col = lax.broadcasted_iota(jnp.int32, (TG, E_), 1)
    m1 = jnp.max(sig, axis=1, keepdims=True)
    i1 = jnp.min(jnp.where(sig == m1, col, E_), axis=1, keepdims=True)
    sig2 = jnp.where(col == i1, -1.0, sig)
    m2 = jnp.max(sig2, axis=1, keepdims=True)
    i2 = jnp.min(jnp.where(sig2 == m2, col, E_), axis=1, keepdims=True)
    g_ref[...] = jnp.concatenate([m1, m2], axis=1)
    i_ref[...] = jnp.concatenate([i1, i2], axis=1)

    o1 = (col == i1).astype(jnp.float32)
    o2 = (col == i2).astype(jnp.float32)
    o = o1 + o2
    row = lax.broadcasted_iota(jnp.int32, (TG, TG), 0)
    cc = lax.broadcasted_iota(jnp.int32, (TG, TG), 1)
    tril = (row > cc).astype(jnp.float32)
    cex = jnp.dot(tril, o, preferred_element_type=jnp.float32)
    prev = jnp.where(s == 0, 0.0, c_ref[...])
    r1 = jnp.sum((cex + prev) * o1, axis=1, keepdims=True)
    r2 = jnp.sum((cex + prev) * o2, axis=1, keepdims=True)
    r_ref[...] = jnp.concatenate([r1, r2], axis=1).astype(jnp.int32)
    c_ref[...] = prev + jnp.sum(o, axis=0, keepdims=True)


def _gate(xf, wg):
    return pl.pallas_call(
        _gate_kernel,
        grid=(T_ // TG,),
        in_specs=[
            pl.BlockSpec((TG, D_), lambda s: (s, 0)),
            pl.BlockSpec((D_, E_), lambda s: (0, 0)),
        ],
        out_specs=[
            pl.BlockSpec((TG, K_), lambda s: (s, 0)),
            pl.BlockSpec((TG, K_), lambda s: (s, 0)),
            pl.BlockSpec((TG, K_), lambda s: (s, 0)),
            pl.BlockSpec((1, E_), lambda s: (0, 0)),
        ],
        out_shape=[
            jax.ShapeDtypeStruct((T_, K_), jnp.float32),
            jax.ShapeDtypeStruct((T_, K_), jnp.int32),
            jax.ShapeDtypeStruct((T_, K_), jnp.int32),
            jax.ShapeDtypeStruct((1, E_), jnp.float32),
        ],
    )(xf, wg)


TM = 512
NTILES = N_ // TM
SMAX = NTILES + E_ - 1


def _gmm_kernel(m_ref, x_ref, w_ref, y_ref):
    s = pl.program_id(0)
    tile = m_ref[0, s]
    first = m_ref[2, s]
    lo = m_ref[3, s]
    hi = m_ref[4, s]
    row = tile * TM + lax.broadcasted_iota(jnp.int32, (TM, 1), 0)
    mask = (row >= lo) & (row < hi)

    @pl.when(hi > lo)
    def _():
        xm = jnp.where(mask, x_ref[...], 0.0)
        acc = jnp.dot(xm, w_ref[0], preferred_element_type=jnp.float32)

        @pl.when(first == 1)
        def _():
            y_ref[...] = acc

        @pl.when(first == 0)
        def _():
            y_ref[...] += acc


def _gmm(meta, xs, we):
    grid_spec = pltpu.PrefetchScalarGridSpec(
        num_scalar_prefetch=1,
        grid=(SMAX,),
        in_specs=[
            pl.BlockSpec((TM, D_), lambda s, m: (m[0, s], 0)),
            pl.BlockSpec((1, D_, D_), lambda s, m: (m[1, s], 0, 0)),
        ],
        out_specs=pl.BlockSpec((TM, D_), lambda s, m: (m[0, s], 0)),
    )
    return pl.pallas_call(
        _gmm_kernel,
        grid_spec=grid_spec,
        out_shape=jax.ShapeDtypeStruct((N_, D_), jnp.float32),
    )(meta, xs, we)


def _gmm_metadata(counts):
    ends = jnp.cumsum(counts)
    starts = ends - counts
    nonempty = counts > 0
    t_first = starts // TM
    t_cnt = jnp.where(nonempty, (ends + TM - 1) // TM - t_first, 0)
    base = jnp.cumsum(t_cnt) - t_cnt
    j = jnp.arange(NTILES, dtype=jnp.int32)[None, :]
    valid = j < t_cnt[:, None]
    step = jnp.where(valid, base[:, None] + j, SMAX)
    tile = t_first[:, None] + j
    lo = jnp.maximum(starts[:, None], tile * TM)
    hi = jnp.minimum(ends[:, None], (tile + 1) * TM)
    gid = jnp.broadcast_to(jnp.arange(E_, dtype=jnp.int32)[:, None], (E_, NTILES))
    last_g = jnp.max(jnp.where(nonempty, jnp.arange(E_, dtype=jnp.int32), -1))

    tile_a = jnp.full((SMAX,), NTILES - 1, jnp.int32).at[step.ravel()].set(
        tile.ravel().astype(jnp.int32), mode="drop")
    gid_a = jnp.full((SMAX,), last_g, jnp.int32).at[step.ravel()].set(
        gid.ravel(), mode="drop")
    lo_a = jnp.full((SMAX,), 1, jnp.int32).at[step.ravel()].set(
        lo.ravel().astype(jnp.int32), mode="drop")
    hi_a = jnp.full((SMAX,), 0, jnp.int32).at[step.ravel()].set(
        hi.ravel().astype(jnp.int32), mode="drop")
    first_a = jnp.concatenate(
        [jnp.ones((1,), jnp.int32),
         (tile_a[1:] != tile_a[:-1]).astype(jnp.int32)])
    return jnp.stack([tile_a, gid_a, first_a, lo_a, hi_a])


_NC, _NS = 2, 16
NW = _NC * _NS
PW = N_ // NW
SB = 64
_mesh = functools.partial(
    plsc.VectorSubcoreMesh, core_axis_name="c", subcore_axis_name="s")


def _dispatch(x2d, slot_flat):
    @functools.partial(
        pl.kernel,
        mesh=_mesh(),
        out_type=jax.ShapeDtypeStruct((N_, D_), jnp.float32),
        scratch_types=[
            pltpu.VMEM((SB,), jnp.int32),
            pltpu.VMEM((SB,), jnp.int32),
            pltpu.VMEM((SB, D_), jnp.float32),
            pltpu.SemaphoreType.DMA,
            pltpu.SemaphoreType.DMA,
        ],
    )
    def disp(x_hbm, slot_hbm, xs_hbm, tok_v, slot_v, rows_v, sem_g, sem_s):
        wid = lax.axis_index("s") * _NC + lax.axis_index("c")
        base = wid * PW
        for b in range(PW // SB):
            pb = base + b * SB
            pltpu.sync_copy(slot_hbm.at[pl.ds(pb, SB)], slot_v)
            for c in range(SB // 16):
                base16 = jnp.full((16,), pb + c * 16, jnp.int32)
                pair16 = base16 + lax.iota(jnp.int32, 16)
                tok_v[pl.ds(c * 16, 16)] = lax.shift_right_logical(
                    pair16, jnp.full((16,), 1, jnp.int32))
            pltpu.async_copy(x_hbm.at[tok_v], rows_v, sem_g).wait()
            pltpu.async_copy(rows_v, xs_hbm.at[slot_v], sem_s).wait()

    return disp(x2d, slot_flat)


PT = T_ // NW
SB2 = 32


def _splat(vec16, lane16):
    dnums = lax.GatherDimensionNumbers(
        offset_dims=(), collapsed_slice_dims=(0,), start_index_map=(0,))
    return lax.gather(vec16, lane16[:, None], dnums, slice_sizes=(1,),
                      mode=lax.GatherScatterMode.PROMISE_IN_BOUNDS)


def _combine(ys, s0, s1, g0, g1):
    @functools.partial(
        pl.kernel,
        mesh=_mesh(),
        out_type=jax.ShapeDtypeStruct((T_, D_), jnp.float32),
        scratch_types=[
            pltpu.VMEM((SB2,), jnp.int32),
            pltpu.VMEM((SB2,), jnp.int32),
            pltpu.VMEM((SB2,), jnp.float32),
            pltpu.VMEM((SB2,), jnp.float32),
            pltpu.VMEM((SB2, D_), jnp.float32),
            pltpu.VMEM((SB2, D_), jnp.float32),
            pltpu.VMEM((SB2, D_), jnp.float32),
            pltpu.SemaphoreType.DMA,
        ],
    )
    def comb(ys_hbm, s0_hbm, s1_hbm, g0_hbm, g1_hbm, out_hbm,
             s0_v, s1_v, g0_v, g1_v, r0_v, r1_v, o_v, sem):
        wid = lax.axis_index("s") * _NC + lax.axis_index("c")
        tb0 = wid * PT
        for b in range(PT // SB2):
            tb = tb0 + b * SB2
            pltpu.sync_copy(s0_hbm.at[pl.ds(tb, SB2)], s0_v)
            pltpu.sync_copy(s1_hbm.at[pl.ds(tb, SB2)], s1_v)
            pltpu.sync_copy(g0_hbm.at[pl.ds(tb, SB2)], g0_v)
            pltpu.sync_copy(g1_hbm.at[pl.ds(tb, SB2)], g1_v)
            pltpu.async_copy(ys_hbm.at[s0_v], r0_v, sem).wait()
            pltpu.async_copy(ys_hbm.at[s1_v], r1_v, sem).wait()

            def row_body(i, _):
                grp = lax.shift_left(lax.shift_right_logical(i, 4), 4)
                lane = jnp.full((16,), i, jnp.int32) & jnp.full((16,), 15, jnp.int32)
                ga = _splat(g0_v[pl.ds(grp, 16)], lane)
                gb = _splat(g1_v[pl.ds(grp, 16)], lane)
                for c in range(D_ // 16):
                    sl = pl.ds(c * 16, 16)
                    o_v[i, sl] = ga * r0_v[i, sl] + gb * r1_v[i, sl]
                return 0

            lax.fori_loop(0, SB2, row_body, 0)
            pltpu.sync_copy(o_v, out_hbm.at[pl.ds(tb, SB2)])

    return comb(ys, s0, s1, g0, g1)


def kernel(x, Wg, We):
    xf = x.reshape(T_, D_)
    gates, ids, ranks, counts = _gate(xf, Wg)
    counts_i = counts[0].astype(jnp.int32)
    offsets = jnp.cumsum(counts_i) - counts_i
    slot = jnp.take(offsets, ids, axis=0) + ranks
    meta = _gmm_metadata(counts_i)
    xs = _dispatch(xf, slot.reshape(-1))
    ys = _gmm(meta, xs, We)
    out = _combine(ys, slot[:, 0], slot[:, 1], gates[:, 0], gates[:, 1])
    return out.reshape(B_, L_, D_)
